# baseline (speedup 1.0000x reference)
import functools

import jax
import jax.numpy as jnp
from jax.experimental import pallas as pl

D_MODEL = 512
SQUARES = 64
BB = 64


def _fused_body(pieces_ref, positions_ref, mp_ref, ptab_ref, qtab_ref,
                w_ref, b_ref, out_ref):
    rows = BB * SQUARES
    mp = mp_ref[...].reshape(rows, SQUARES)
    acc = jnp.dot(mp, w_ref[...], preferred_element_type=jnp.float32)

    p = pieces_ref[...][:, :, None]
    oh_p = (p == jax.lax.broadcasted_iota(jnp.int32, (BB, SQUARES, 16), 2)
            ).astype(jnp.float32).reshape(rows, 16)
    acc += jnp.dot(oh_p, ptab_ref[...], preferred_element_type=jnp.float32)

    q = positions_ref[...][:, :, None]
    oh_q = (q == jax.lax.broadcasted_iota(jnp.int32, (BB, SQUARES, SQUARES), 2)
            ).astype(jnp.float32).reshape(rows, SQUARES)
    acc += jnp.dot(oh_q, qtab_ref[...], preferred_element_type=jnp.float32)

    acc += b_ref[...]
    out_ref[...] = acc.reshape(BB, SQUARES, D_MODEL)


@jax.jit
def kernel(pieces, positions, move_potentials, piece_table, pos_table, W, b):
    batch = pieces.shape[0]
    grid = batch // BB
    pieces = pieces.astype(jnp.int32)
    positions = positions.astype(jnp.int32)
    move_potentials = move_potentials.astype(jnp.bfloat16)
    W = W.astype(jnp.bfloat16)
    ptab = jnp.zeros((16, D_MODEL), jnp.float32).at[:13].set(piece_table)
    b2 = b.reshape(1, D_MODEL)

    out = pl.pallas_call(
        _fused_body,
        grid=(grid,),
        in_specs=[
            pl.BlockSpec((BB, SQUARES), lambda i: (i, 0)),
            pl.BlockSpec((BB, SQUARES), lambda i: (i, 0)),
            pl.BlockSpec((BB, SQUARES, SQUARES), lambda i: (i, 0, 0)),
            pl.BlockSpec((16, D_MODEL), lambda i: (0, 0)),
            pl.BlockSpec((SQUARES, D_MODEL), lambda i: (0, 0)),
            pl.BlockSpec((SQUARES, D_MODEL), lambda i: (0, 0)),
            pl.BlockSpec((1, D_MODEL), lambda i: (0, 0)),
        ],
        out_specs=pl.BlockSpec((BB, SQUARES, D_MODEL), lambda i: (i, 0, 0)),
        out_shape=jax.ShapeDtypeStruct((batch, SQUARES, D_MODEL), jnp.float32),
    )(pieces, positions, move_potentials, ptab, pos_table, W, b2)
    return out


# speedup vs baseline: 1.0373x; 1.0373x over previous
import functools

import jax
import jax.numpy as jnp
from jax.experimental import pallas as pl

D_MODEL = 512
SQUARES = 64
BB = 64


def _fused_body(pieces_ref, positions_ref, mp_ref, ptab_ref, qtab_ref,
                w_ref, b_ref, out_ref):
    rows = BB * SQUARES
    mp = mp_ref[...].reshape(rows, SQUARES).astype(jnp.bfloat16)
    acc = jnp.dot(mp, w_ref[...].astype(jnp.bfloat16),
                  preferred_element_type=jnp.float32)

    p = pieces_ref[...][:, :, None]
    oh_p = (p == jax.lax.broadcasted_iota(jnp.int32, (BB, SQUARES, 16), 2)
            ).astype(jnp.bfloat16).reshape(rows, 16)
    acc += jnp.dot(oh_p, ptab_ref[...].astype(jnp.bfloat16),
                   preferred_element_type=jnp.float32)

    q = positions_ref[...][:, :, None]
    oh_q = (q == jax.lax.broadcasted_iota(jnp.int32, (BB, SQUARES, SQUARES), 2)
            ).astype(jnp.bfloat16).reshape(rows, SQUARES)
    acc += jnp.dot(oh_q, qtab_ref[...].astype(jnp.bfloat16),
                   preferred_element_type=jnp.float32)

    acc += b_ref[...]
    out_ref[...] = acc.reshape(BB, SQUARES, D_MODEL)


@jax.jit
def kernel(pieces, positions, move_potentials, piece_table, pos_table, W, b):
    batch = pieces.shape[0]
    grid = batch // BB
    pieces = pieces.astype(jnp.int32)
    positions = positions.astype(jnp.int32)
    ptab = jnp.zeros((16, D_MODEL), jnp.float32).at[:13].set(piece_table)
    b2 = b.reshape(1, D_MODEL)

    out = pl.pallas_call(
        _fused_body,
        grid=(grid,),
        in_specs=[
            pl.BlockSpec((BB, SQUARES), lambda i: (i, 0)),
            pl.BlockSpec((BB, SQUARES), lambda i: (i, 0)),
            pl.BlockSpec((BB, SQUARES, SQUARES), lambda i: (i, 0, 0)),
            pl.BlockSpec((16, D_MODEL), lambda i: (0, 0)),
            pl.BlockSpec((SQUARES, D_MODEL), lambda i: (0, 0)),
            pl.BlockSpec((SQUARES, D_MODEL), lambda i: (0, 0)),
            pl.BlockSpec((1, D_MODEL), lambda i: (0, 0)),
        ],
        out_specs=pl.BlockSpec((BB, SQUARES, D_MODEL), lambda i: (i, 0, 0)),
        out_shape=jax.ShapeDtypeStruct((batch, SQUARES, D_MODEL), jnp.float32),
    )(pieces, positions, move_potentials, ptab, pos_table, W, b2)
    return out
